# baseline (speedup 1.0000x reference)
import functools

import numpy as np

import jax
import jax.numpy as jnp
from jax import lax
from jax.experimental import pallas as pl
from jax.experimental.pallas import tpu as pltpu
from jax.experimental.pallas import tpu_sc as plsc

EMBED = 256
HEADS = 8
LEVELS = 4
POINTS = 8
HEAD_DIM = 32
LP = LEVELS * POINTS
NQ = 10000
NV = 21760
NPAIR = NQ * HEADS
NCORNER = LP * 4
NGATH = LP
_LVL_W = np.array([128, 64, 32, 16], dtype=np.int32)
_STARTS = np.array([0, 16384, 20480, 21504], dtype=np.int32)

_ch = np.arange(EMBED)
_l_of = (_ch // POINTS) % LEVELS
_WC_I = np.asarray(_LVL_W[_l_of], np.int32)[None]
_WC_F = _WC_I.astype(np.float32)
_START_C = np.asarray(_STARTS[_l_of], np.int32)[None]
_HOFF_C = np.asarray((_ch // LP) * NV, np.int32)[None]
_S_LVL = np.zeros((LEVELS, EMBED), np.float32)
_S_LVL[_l_of, _ch] = 1.0
_HW = np.array([16384, 4096, 1024, 256], dtype=np.int32)
_PB_C = np.asarray(_LVL_W[_l_of] + 1 + (_ch // LP) * _HW[_l_of], np.int32)[None]
_PMAX_C = np.asarray(8 * _HW[_l_of] + _LVL_W[_l_of], np.int32)[None]
_PERM = np.concatenate([2 * np.arange(EMBED), 2 * np.arange(EMBED) + 1])
_t = np.arange(EMBED) % HEAD_DIM
_PERM_SC = ((np.arange(EMBED) // HEAD_DIM) * HEAD_DIM
            + np.where(_t < 16, 2 * _t, 2 * (_t - 16) + 1))

BQ = 1000
BV = 680


def _vproj_body(v_ref, wvT_ref, bv_ref, out_ref):
    res = jnp.dot(v_ref[...], wvT_ref[...], preferred_element_type=jnp.float32, precision=lax.Precision.HIGHEST)
    res = (res + bv_ref[...]).astype(jnp.bfloat16)
    for h in range(HEADS):
        out_ref[h] = res[:, h * HEAD_DIM:(h + 1) * HEAD_DIM]


def _locs_body(q_ref, refx_ref, refy_ref, wsoT_ref, bso_ref, wawT_ref, baw_ref,
               slvl_ref, wcf_ref, wci_ref, pb_ref, pmax_ref, idx_ref, wgt_ref):
    wc_f = wcf_ref[...]
    wc_i = wci_ref[...]
    pb_c = pb_ref[...]
    pmax_c = pmax_ref[...]

    q = q_ref[...]
    so = jnp.dot(q, wsoT_ref[...], preferred_element_type=jnp.float32) + bso_ref[...]
    aw = jnp.dot(q, wawT_ref[...], preferred_element_type=jnp.float32) + baw_ref[...]
    aw3 = aw.reshape(BQ, HEADS, LP)
    aw3 = aw3 - jnp.max(aw3, axis=-1, keepdims=True)
    e = jnp.exp(aw3)
    aw = (e / jnp.sum(e, axis=-1, keepdims=True)).reshape(BQ, EMBED)

    rx = jnp.dot(refx_ref[...], slvl_ref[...], preferred_element_type=jnp.float32, precision=lax.Precision.HIGHEST)
    ry = jnp.dot(refy_ref[...], slvl_ref[...], preferred_element_type=jnp.float32, precision=lax.Precision.HIGHEST)
    x = rx * wc_f + so[:, :EMBED] - 0.5
    y = ry * wc_f + so[:, EMBED:] - 0.5
    x = jnp.clip(x, -2.0, wc_f + 1.0)
    y = jnp.clip(y, -2.0, wc_f + 1.0)
    x0 = jnp.floor(x)
    y0 = jnp.floor(y)
    fx = x - x0
    fy = y - y0
    ix = x0.astype(jnp.int32)
    iy = y0.astype(jnp.int32)

    vx0 = ((ix >= 0) & (ix < wc_i)).astype(jnp.float32)
    vx1 = ((ix + 1 >= 0) & (ix + 1 < wc_i)).astype(jnp.float32)
    vy0 = ((iy >= 0) & (iy < wc_i)).astype(jnp.float32)
    vy1 = ((iy + 1 >= 0) & (iy + 1 < wc_i)).astype(jnp.float32)
    w00 = (1.0 - fx) * (1.0 - fy) * aw * (vx0 * vy0)
    w10 = fx * (1.0 - fy) * aw * (vx1 * vy0)
    w01 = (1.0 - fx) * fy * aw * (vx0 * vy1)
    w11 = fx * fy * aw * (vx1 * vy1)
    pidx = jnp.clip(pb_c + iy * wc_i + ix, 0, pmax_c)
    idx_ref[...] = pidx
    wgt_ref[0] = w00
    wgt_ref[1] = w10
    wgt_ref[2] = w01
    wgt_ref[3] = w11


def _out_body(sc_ref, q_ref, woT_ref, bo_ref, out_ref):
    sc = jnp.concatenate([sc_ref[h] for h in range(HEADS)], axis=-1)
    res = jnp.dot(sc, woT_ref[...], preferred_element_type=jnp.float32, precision=lax.Precision.HIGHEST)
    out_ref[...] = res + bo_ref[...] + q_ref[...]


NW = 32
CHUNK = 16
NW_HI = 4
NCHUNK_HI = 158
NCHUNK_LO = 156


def _sc_gather_fn():
    mesh = plsc.VectorSubcoreMesh(core_axis_name="c", subcore_axis_name="s")

    @functools.partial(
        pl.kernel,
        mesh=mesh,
        out_type=jax.ShapeDtypeStruct((NPAIR, HEAD_DIM), jnp.float32),
        scratch_types=[
            pltpu.VMEM((2, CHUNK, NGATH), jnp.int32),
            pltpu.VMEM((2, 4, CHUNK, LP), jnp.float32),
            pltpu.VMEM((2, CHUNK, NGATH, 4, HEAD_DIM), jnp.bfloat16),
            pltpu.VMEM((2, CHUNK, HEAD_DIM), jnp.float32),
            pltpu.SemaphoreType.DMA,
            pltpu.SemaphoreType.DMA,
            pltpu.SemaphoreType.DMA,
            pltpu.SemaphoreType.DMA,
            pltpu.SemaphoreType.DMA,
            pltpu.SemaphoreType.DMA,
        ],
        compiler_params=pltpu.CompilerParams(use_tc_tiling_on_sc=False,
                                             needs_layout_passes=False),
    )
    def sc_gather(vp0_hbm, vp1_hbm, vp2_hbm, vp3_hbm, idx_hbm, wgt_hbm, out_hbm,
                  idx_v, wgt_v, rows_v, out_v,
                  sem_io0, sem_io1, sem_g0, sem_g1, sem_o0, sem_o1):
        vp_hbm = (vp0_hbm, vp1_hbm, vp2_hbm, vp3_hbm)
        sem_io = (sem_io0, sem_io1)
        sem_g = (sem_g0, sem_g1)
        sem_o = (sem_o0, sem_o1)
        wid = lax.axis_index("s") * 2 + lax.axis_index("c")
        hi = wid < NW_HI
        base0 = jnp.where(
            hi, wid * (CHUNK * NCHUNK_HI),
            NW_HI * CHUNK * NCHUNK_HI + (wid - NW_HI) * (CHUNK * NCHUNK_LO))
        nchunk = jnp.where(hi, NCHUNK_HI, NCHUNK_LO)

        def io_src(c):
            return pl.ds(base0 + c * CHUNK, CHUNK)

        def hq(c):
            p = base0 + c * CHUNK
            h = jnp.int32(0)
            for k in range(1, HEADS):
                h = h + (p >= k * NQ).astype(jnp.int32)
            return h, p - h * NQ

        def prefetch_io(c, b):
            h, q0 = hq(c)
            pltpu.async_copy(
                idx_hbm.at[pl.ds(q0, CHUNK), pl.ds(h * LP, LP)],
                idx_v.at[b], sem_io[b])
            pltpu.async_copy(
                wgt_hbm.at[:, pl.ds(q0, CHUNK), pl.ds(h * LP, LP)],
                wgt_v.at[b], sem_io[b])

        def wait_io(b):
            pltpu.make_async_copy(
                idx_hbm.at[pl.ds(0, CHUNK), pl.ds(0, LP)],
                idx_v.at[b], sem_io[b]).wait()
            pltpu.make_async_copy(
                wgt_hbm.at[:, pl.ds(0, CHUNK), pl.ds(0, LP)],
                wgt_v.at[b], sem_io[b]).wait()

        def fire_gathers(b):
            for j in range(CHUNK):
                for l in range(LEVELS):
                    pltpu.async_copy(
                        vp_hbm[l].at[idx_v.at[b, j, pl.ds(l * POINTS, POINTS)]],
                        rows_v.at[b, j, pl.ds(l * POINTS, POINTS)], sem_g[b])

        def wait_gathers(b):
            for j in range(CHUNK):
                for l in range(LEVELS):
                    pltpu.make_async_copy(
                        vp_hbm[l].at[idx_v.at[b, j, pl.ds(l * POINTS, POINTS)]],
                        rows_v.at[b, j, pl.ds(l * POINTS, POINTS)],
                        sem_g[b]).wait()

        def step(c, b):
            @pl.when(c + 1 < nchunk)
            def _():
                wait_io(1 - b)
                fire_gathers(1 - b)

            wait_gathers(b)

            @pl.when(c >= 2)
            def _():
                pltpu.make_async_copy(out_v.at[b], out_hbm.at[io_src(0)],
                                      sem_o[b]).wait()

            for j in range(CHUNK):
                def g_body(g, accs):
                    ae, ao = accs
                    ws = [wgt_v[b, s, j, pl.ds(g * 16, 16)]
                          for s in range(4)]
                    for i in range(16):
                        k = g * 16 + i
                        for s in range(4):
                            px = rows_v[b, j, k, s, :]
                            e, o = plsc.unpack(
                                px, format=plsc.PackFormat.INTERLEAVED)
                            ae = ae + ws[s][i] * e
                            ao = ao + ws[s][i] * o
                    return (ae, ao)
                ae, ao = lax.fori_loop(
                    0, NGATH // 16, g_body,
                    (jnp.zeros((16,), jnp.float32), jnp.zeros((16,), jnp.float32)))
                out_v[b, j, pl.ds(0, 16)] = ae
                out_v[b, j, pl.ds(16, 16)] = ao

            @pl.when(c + 2 < nchunk)
            def _():
                prefetch_io(c + 2, b)

            pltpu.async_copy(out_v.at[b], out_hbm.at[io_src(c)], sem_o[b])

        prefetch_io(0, 0)
        prefetch_io(1, 1)
        wait_io(0)
        fire_gathers(0)

        def pair_body(c2, carry):
            step(2 * c2, 0)
            step(2 * c2 + 1, 1)
            return carry

        lax.fori_loop(0, nchunk // 2, pair_body, 0)

        pltpu.make_async_copy(out_v.at[0], out_hbm.at[io_src(0)], sem_o0).wait()
        pltpu.make_async_copy(out_v.at[1], out_hbm.at[io_src(0)], sem_o1).wait()

    return sc_gather


def kernel(query, value, reference_points, spatial_shapes, level_start_index,
           W_so, b_so, W_aw, b_aw, W_v, b_v, W_o, b_o):
    q2 = query[0]
    v2 = value[0]
    rp = reference_points[0]
    refx = rp[:, :, 0]
    refy = rp[:, :, 1]

    vt8 = pl.pallas_call(
        _vproj_body,
        grid=(NV // BV,),
        in_specs=[
            pl.BlockSpec((BV, EMBED), lambda i: (i, 0)),
            pl.BlockSpec((EMBED, EMBED), lambda i: (0, 0)),
            pl.BlockSpec((1, EMBED), lambda i: (0, 0)),
        ],
        out_specs=pl.BlockSpec((HEADS, BV, HEAD_DIM), lambda i: (0, i, 0)),
        out_shape=jax.ShapeDtypeStruct((HEADS, NV, HEAD_DIM), jnp.bfloat16),
    )(v2, W_v.T, b_v[None])
    tables = []
    for l in range(LEVELS):
        w = int(_LVL_W[l])
        hw = int(_HW[l])
        start = int(_STARTS[l])
        fl = vt8[:, start:start + hw, :].reshape(HEADS * hw, HEAD_DIM)
        flp = jnp.pad(fl, ((w + 1, w + 1), (0, 0)))
        n = HEADS * hw + w + 1
        tables.append(jnp.stack(
            [flp[0:n], flp[1:n + 1], flp[w:w + n], flp[w + 1:w + n + 1]],
            axis=1))

    wsoT = W_so.T[:, _PERM]
    bso = b_so[_PERM][None]
    idx8, wgt8 = pl.pallas_call(
        _locs_body,
        grid=(NQ // BQ,),
        in_specs=[
            pl.BlockSpec((BQ, EMBED), lambda i: (i, 0)),
            pl.BlockSpec((BQ, LEVELS), lambda i: (i, 0)),
            pl.BlockSpec((BQ, LEVELS), lambda i: (i, 0)),
            pl.BlockSpec((EMBED, 2 * EMBED), lambda i: (0, 0)),
            pl.BlockSpec((1, 2 * EMBED), lambda i: (0, 0)),
            pl.BlockSpec((EMBED, EMBED), lambda i: (0, 0)),
            pl.BlockSpec((1, EMBED), lambda i: (0, 0)),
            pl.BlockSpec((LEVELS, EMBED), lambda i: (0, 0)),
            pl.BlockSpec((1, EMBED), lambda i: (0, 0)),
            pl.BlockSpec((1, EMBED), lambda i: (0, 0)),
            pl.BlockSpec((1, EMBED), lambda i: (0, 0)),
            pl.BlockSpec((1, EMBED), lambda i: (0, 0)),
        ],
        out_specs=[
            pl.BlockSpec((BQ, EMBED), lambda i: (i, 0)),
            pl.BlockSpec((4, BQ, EMBED), lambda i: (0, i, 0)),
        ],
        out_shape=[
            jax.ShapeDtypeStruct((NQ, EMBED), jnp.int32),
            jax.ShapeDtypeStruct((4, NQ, EMBED), jnp.float32),
        ],
    )(q2, refx, refy, wsoT, bso, W_aw.T, b_aw[None],
      jnp.asarray(_S_LVL), jnp.asarray(_WC_F), jnp.asarray(_WC_I),
      jnp.asarray(_PB_C), jnp.asarray(_PMAX_C))
    idx2 = idx8
    wgt2 = wgt8

    sc_out = _sc_gather_fn()(*tables, idx2, wgt2)
    sc3 = sc_out.reshape(HEADS, NQ, HEAD_DIM)

    out = pl.pallas_call(
        _out_body,
        grid=(NQ // BQ,),
        in_specs=[
            pl.BlockSpec((HEADS, BQ, HEAD_DIM), lambda i: (0, i, 0)),
            pl.BlockSpec((BQ, EMBED), lambda i: (i, 0)),
            pl.BlockSpec((EMBED, EMBED), lambda i: (0, 0)),
            pl.BlockSpec((1, EMBED), lambda i: (0, 0)),
        ],
        out_specs=pl.BlockSpec((BQ, EMBED), lambda i: (i, 0)),
        out_shape=jax.ShapeDtypeStruct((NQ, EMBED), jnp.float32),
    )(sc3, q2, W_o.T[_PERM_SC], b_o[None])
    return out[None]


# speedup vs baseline: 1.0293x; 1.0293x over previous
import functools

import numpy as np

import jax
import jax.numpy as jnp
from jax import lax
from jax.experimental import pallas as pl
from jax.experimental.pallas import tpu as pltpu
from jax.experimental.pallas import tpu_sc as plsc

EMBED = 256
HEADS = 8
LEVELS = 4
POINTS = 8
HEAD_DIM = 32
LP = LEVELS * POINTS
NQ = 10000
NV = 21760
NPAIR = NQ * HEADS
NCORNER = LP * 4
NGATH = LP
_LVL_W = np.array([128, 64, 32, 16], dtype=np.int32)
_STARTS = np.array([0, 16384, 20480, 21504], dtype=np.int32)

_ch = np.arange(EMBED)
_l_of = (_ch // POINTS) % LEVELS
_WC_I = np.asarray(_LVL_W[_l_of], np.int32)[None]
_WC_F = _WC_I.astype(np.float32)
_START_C = np.asarray(_STARTS[_l_of], np.int32)[None]
_HOFF_C = np.asarray((_ch // LP) * NV, np.int32)[None]
_S_LVL = np.zeros((LEVELS, EMBED), np.float32)
_S_LVL[_l_of, _ch] = 1.0
_HW = np.array([16384, 4096, 1024, 256], dtype=np.int32)
_PB_C = np.asarray(_LVL_W[_l_of] + 1 + (_ch // LP) * _HW[_l_of], np.int32)[None]
_PMAX_C = np.asarray(8 * _HW[_l_of] + _LVL_W[_l_of], np.int32)[None]
_PERM = np.concatenate([2 * np.arange(EMBED), 2 * np.arange(EMBED) + 1])
_t = np.arange(EMBED) % HEAD_DIM
_PERM_SC = ((np.arange(EMBED) // HEAD_DIM) * HEAD_DIM
            + np.where(_t < 16, 2 * _t, 2 * (_t - 16) + 1))

BQ = 1000
BV = 680


def _vproj_body(v_ref, wvT_ref, bv_ref, out_ref):
    res = jnp.dot(v_ref[...], wvT_ref[...], preferred_element_type=jnp.float32, precision=lax.Precision.HIGHEST)
    res = (res + bv_ref[...]).astype(jnp.bfloat16)
    for h in range(HEADS):
        out_ref[h] = res[:, h * HEAD_DIM:(h + 1) * HEAD_DIM]


def _locs_body(q_ref, refx_ref, refy_ref, wsoT_ref, bso_ref, wawT_ref, baw_ref,
               slvl_ref, wcf_ref, wci_ref, pb_ref, pmax_ref, idx_ref, wgt_ref):
    wc_f = wcf_ref[...]
    wc_i = wci_ref[...]
    pb_c = pb_ref[...]
    pmax_c = pmax_ref[...]

    q = q_ref[...]
    so = jnp.dot(q, wsoT_ref[...], preferred_element_type=jnp.float32) + bso_ref[...]
    aw = jnp.dot(q, wawT_ref[...], preferred_element_type=jnp.float32) + baw_ref[...]
    aw3 = aw.reshape(BQ, HEADS, LP)
    aw3 = aw3 - jnp.max(aw3, axis=-1, keepdims=True)
    e = jnp.exp(aw3)
    aw = (e / jnp.sum(e, axis=-1, keepdims=True)).reshape(BQ, EMBED)

    rx = jnp.dot(refx_ref[...], slvl_ref[...], preferred_element_type=jnp.float32, precision=lax.Precision.HIGHEST)
    ry = jnp.dot(refy_ref[...], slvl_ref[...], preferred_element_type=jnp.float32, precision=lax.Precision.HIGHEST)
    x = rx * wc_f + so[:, :EMBED] - 0.5
    y = ry * wc_f + so[:, EMBED:] - 0.5
    x = jnp.clip(x, -2.0, wc_f + 1.0)
    y = jnp.clip(y, -2.0, wc_f + 1.0)
    x0 = jnp.floor(x)
    y0 = jnp.floor(y)
    fx = x - x0
    fy = y - y0
    ix = x0.astype(jnp.int32)
    iy = y0.astype(jnp.int32)

    vx0 = ((ix >= 0) & (ix < wc_i)).astype(jnp.float32)
    vx1 = ((ix + 1 >= 0) & (ix + 1 < wc_i)).astype(jnp.float32)
    vy0 = ((iy >= 0) & (iy < wc_i)).astype(jnp.float32)
    vy1 = ((iy + 1 >= 0) & (iy + 1 < wc_i)).astype(jnp.float32)
    w00 = (1.0 - fx) * (1.0 - fy) * aw * (vx0 * vy0)
    w10 = fx * (1.0 - fy) * aw * (vx1 * vy0)
    w01 = (1.0 - fx) * fy * aw * (vx0 * vy1)
    w11 = fx * fy * aw * (vx1 * vy1)
    pidx = jnp.clip(pb_c + iy * wc_i + ix, 0, pmax_c)
    idx_ref[...] = pidx
    wgt_ref[0] = w00
    wgt_ref[1] = w10
    wgt_ref[2] = w01
    wgt_ref[3] = w11


def _out_body(sc_ref, q_ref, woT_ref, bo_ref, out_ref):
    sc = jnp.concatenate([sc_ref[h] for h in range(HEADS)], axis=-1)
    res = jnp.dot(sc, woT_ref[...], preferred_element_type=jnp.float32, precision=lax.Precision.HIGHEST)
    out_ref[...] = res + bo_ref[...] + q_ref[...]


NW = 32
CHUNK = 8
NW_HI = 8
NCHUNK_HI = 314
NCHUNK_LO = 312


def _sc_gather_fn():
    mesh = plsc.VectorSubcoreMesh(core_axis_name="c", subcore_axis_name="s")

    @functools.partial(
        pl.kernel,
        mesh=mesh,
        out_type=jax.ShapeDtypeStruct((NPAIR, HEAD_DIM), jnp.float32),
        scratch_types=[
            pltpu.VMEM((2, CHUNK, NGATH), jnp.int32),
            pltpu.VMEM((2, 4, CHUNK, LP), jnp.float32),
            pltpu.VMEM((2, CHUNK, NGATH, 4, HEAD_DIM), jnp.bfloat16),
            pltpu.VMEM((2, CHUNK, HEAD_DIM), jnp.float32),
            pltpu.SemaphoreType.DMA,
            pltpu.SemaphoreType.DMA,
            pltpu.SemaphoreType.DMA,
            pltpu.SemaphoreType.DMA,
            pltpu.SemaphoreType.DMA,
            pltpu.SemaphoreType.DMA,
        ],
        compiler_params=pltpu.CompilerParams(use_tc_tiling_on_sc=False,
                                             needs_layout_passes=False),
    )
    def sc_gather(vp0_hbm, vp1_hbm, vp2_hbm, vp3_hbm, idx_hbm, wgt_hbm, out_hbm,
                  idx_v, wgt_v, rows_v, out_v,
                  sem_io0, sem_io1, sem_g0, sem_g1, sem_o0, sem_o1):
        vp_hbm = (vp0_hbm, vp1_hbm, vp2_hbm, vp3_hbm)
        sem_io = (sem_io0, sem_io1)
        sem_g = (sem_g0, sem_g1)
        sem_o = (sem_o0, sem_o1)
        wid = lax.axis_index("s") * 2 + lax.axis_index("c")
        hi = wid < NW_HI
        base0 = jnp.where(
            hi, wid * (CHUNK * NCHUNK_HI),
            NW_HI * CHUNK * NCHUNK_HI + (wid - NW_HI) * (CHUNK * NCHUNK_LO))
        nchunk = jnp.where(hi, NCHUNK_HI, NCHUNK_LO)

        def io_src(c):
            return pl.ds(base0 + c * CHUNK, CHUNK)

        def hq(c):
            p = base0 + c * CHUNK
            h = jnp.int32(0)
            for k in range(1, HEADS):
                h = h + (p >= k * NQ).astype(jnp.int32)
            return h, p - h * NQ

        def prefetch_io(c, b):
            h, q0 = hq(c)
            pltpu.async_copy(
                idx_hbm.at[pl.ds(q0, CHUNK), pl.ds(h * LP, LP)],
                idx_v.at[b], sem_io[b])
            pltpu.async_copy(
                wgt_hbm.at[:, pl.ds(q0, CHUNK), pl.ds(h * LP, LP)],
                wgt_v.at[b], sem_io[b])

        def wait_io(b):
            pltpu.make_async_copy(
                idx_hbm.at[pl.ds(0, CHUNK), pl.ds(0, LP)],
                idx_v.at[b], sem_io[b]).wait()
            pltpu.make_async_copy(
                wgt_hbm.at[:, pl.ds(0, CHUNK), pl.ds(0, LP)],
                wgt_v.at[b], sem_io[b]).wait()

        def fire_gathers(b):
            for j in range(CHUNK):
                for l in range(LEVELS):
                    pltpu.async_copy(
                        vp_hbm[l].at[idx_v.at[b, j, pl.ds(l * POINTS, POINTS)]],
                        rows_v.at[b, j, pl.ds(l * POINTS, POINTS)], sem_g[b])

        def wait_gathers(b):
            for j in range(CHUNK):
                for l in range(LEVELS):
                    pltpu.make_async_copy(
                        vp_hbm[l].at[idx_v.at[b, j, pl.ds(l * POINTS, POINTS)]],
                        rows_v.at[b, j, pl.ds(l * POINTS, POINTS)],
                        sem_g[b]).wait()

        def step(c, b):
            @pl.when(c + 1 < nchunk)
            def _():
                wait_io(1 - b)
                fire_gathers(1 - b)

            wait_gathers(b)

            @pl.when(c >= 2)
            def _():
                pltpu.make_async_copy(out_v.at[b], out_hbm.at[io_src(0)],
                                      sem_o[b]).wait()

            for j in range(CHUNK):
                def g_body(g, accs):
                    ae, ao = accs
                    ws = [wgt_v[b, s, j, pl.ds(g * 16, 16)]
                          for s in range(4)]
                    for i in range(16):
                        k = g * 16 + i
                        for s in range(4):
                            px = rows_v[b, j, k, s, :]
                            e, o = plsc.unpack(
                                px, format=plsc.PackFormat.INTERLEAVED)
                            ae = ae + ws[s][i] * e
                            ao = ao + ws[s][i] * o
                    return (ae, ao)
                ae, ao = lax.fori_loop(
                    0, NGATH // 16, g_body,
                    (jnp.zeros((16,), jnp.float32), jnp.zeros((16,), jnp.float32)))
                out_v[b, j, pl.ds(0, 16)] = ae
                out_v[b, j, pl.ds(16, 16)] = ao

            @pl.when(c + 2 < nchunk)
            def _():
                prefetch_io(c + 2, b)

            pltpu.async_copy(out_v.at[b], out_hbm.at[io_src(c)], sem_o[b])

        prefetch_io(0, 0)
        prefetch_io(1, 1)
        wait_io(0)
        fire_gathers(0)

        def pair_body(c2, carry):
            step(2 * c2, 0)
            step(2 * c2 + 1, 1)
            return carry

        lax.fori_loop(0, nchunk // 2, pair_body, 0)

        pltpu.make_async_copy(out_v.at[0], out_hbm.at[io_src(0)], sem_o0).wait()
        pltpu.make_async_copy(out_v.at[1], out_hbm.at[io_src(0)], sem_o1).wait()

    return sc_gather


def kernel(query, value, reference_points, spatial_shapes, level_start_index,
           W_so, b_so, W_aw, b_aw, W_v, b_v, W_o, b_o):
    q2 = query[0]
    v2 = value[0]
    rp = reference_points[0]
    refx = rp[:, :, 0]
    refy = rp[:, :, 1]

    vt8 = pl.pallas_call(
        _vproj_body,
        grid=(NV // BV,),
        in_specs=[
            pl.BlockSpec((BV, EMBED), lambda i: (i, 0)),
            pl.BlockSpec((EMBED, EMBED), lambda i: (0, 0)),
            pl.BlockSpec((1, EMBED), lambda i: (0, 0)),
        ],
        out_specs=pl.BlockSpec((HEADS, BV, HEAD_DIM), lambda i: (0, i, 0)),
        out_shape=jax.ShapeDtypeStruct((HEADS, NV, HEAD_DIM), jnp.bfloat16),
    )(v2, W_v.T, b_v[None])
    tables = []
    for l in range(LEVELS):
        w = int(_LVL_W[l])
        hw = int(_HW[l])
        start = int(_STARTS[l])
        fl = vt8[:, start:start + hw, :].reshape(HEADS * hw, HEAD_DIM)
        flp = jnp.pad(fl, ((w + 1, w + 1), (0, 0)))
        n = HEADS * hw + w + 1
        tables.append(jnp.stack(
            [flp[0:n], flp[1:n + 1], flp[w:w + n], flp[w + 1:w + n + 1]],
            axis=1))

    wsoT = W_so.T[:, _PERM]
    bso = b_so[_PERM][None]
    idx8, wgt8 = pl.pallas_call(
        _locs_body,
        grid=(NQ // BQ,),
        in_specs=[
            pl.BlockSpec((BQ, EMBED), lambda i: (i, 0)),
            pl.BlockSpec((BQ, LEVELS), lambda i: (i, 0)),
            pl.BlockSpec((BQ, LEVELS), lambda i: (i, 0)),
            pl.BlockSpec((EMBED, 2 * EMBED), lambda i: (0, 0)),
            pl.BlockSpec((1, 2 * EMBED), lambda i: (0, 0)),
            pl.BlockSpec((EMBED, EMBED), lambda i: (0, 0)),
            pl.BlockSpec((1, EMBED), lambda i: (0, 0)),
            pl.BlockSpec((LEVELS, EMBED), lambda i: (0, 0)),
            pl.BlockSpec((1, EMBED), lambda i: (0, 0)),
            pl.BlockSpec((1, EMBED), lambda i: (0, 0)),
            pl.BlockSpec((1, EMBED), lambda i: (0, 0)),
            pl.BlockSpec((1, EMBED), lambda i: (0, 0)),
        ],
        out_specs=[
            pl.BlockSpec((BQ, EMBED), lambda i: (i, 0)),
            pl.BlockSpec((4, BQ, EMBED), lambda i: (0, i, 0)),
        ],
        out_shape=[
            jax.ShapeDtypeStruct((NQ, EMBED), jnp.int32),
            jax.ShapeDtypeStruct((4, NQ, EMBED), jnp.float32),
        ],
    )(q2, refx, refy, wsoT, bso, W_aw.T, b_aw[None],
      jnp.asarray(_S_LVL), jnp.asarray(_WC_F), jnp.asarray(_WC_I),
      jnp.asarray(_PB_C), jnp.asarray(_PMAX_C))
    idx2 = idx8
    wgt2 = wgt8

    sc_out = _sc_gather_fn()(*tables, idx2, wgt2)
    sc3 = sc_out.reshape(HEADS, NQ, HEAD_DIM)

    out = pl.pallas_call(
        _out_body,
        grid=(NQ // BQ,),
        in_specs=[
            pl.BlockSpec((HEADS, BQ, HEAD_DIM), lambda i: (0, i, 0)),
            pl.BlockSpec((BQ, EMBED), lambda i: (i, 0)),
            pl.BlockSpec((EMBED, EMBED), lambda i: (0, 0)),
            pl.BlockSpec((1, EMBED), lambda i: (0, 0)),
        ],
        out_specs=pl.BlockSpec((BQ, EMBED), lambda i: (i, 0)),
        out_shape=jax.ShapeDtypeStruct((NQ, EMBED), jnp.float32),
    )(sc3, q2, W_o.T[_PERM_SC], b_o[None])
    return out[None]
